# baseline (speedup 1.0000x reference)
import functools

import jax
import jax.numpy as jnp
from jax import lax
from jax.experimental import pallas as pl
from jax.experimental.pallas import tpu as pltpu
from jax.experimental.pallas import tpu_sc as plsc

N = 10000
D = 128
H = 128
C = 20
G = 100

NCORES = 2
NSUB = 16
NTILES = NCORES * NSUB
CH = 128
NCPT = 160
NGRP = 10
GC = NCPT // NGRP
NGFAST = 7
CFAST = 1
NCHUNK = 80
NROW = 10112
RPT = NROW // NSUB

_mesh = plsc.VectorSubcoreMesh(core_axis_name="c", subcore_axis_name="s")


def _zero_acc(zeros_hbm, buf, acc, r0):
    pltpu.sync_copy(zeros_hbm, buf)
    for k in range(4):
        pltpu.sync_copy(buf, acc.at[pl.ds(r0 + k * CH, CH)])
    pltpu.sync_copy(buf.at[pl.ds(0, RPT - 4 * CH)],
                    acc.at[pl.ds(r0 + 4 * CH, RPT - 4 * CH)])


@functools.partial(
    pl.kernel,
    out_type=jax.ShapeDtypeStruct((NCORES, NROW, H), jnp.float32),
    mesh=_mesh,
    scratch_types=[
        pltpu.VMEM((GC, CH), jnp.int32),
        pltpu.VMEM((GC, CH), jnp.int32),
        pltpu.VMEM((CH, H), jnp.float32),
        pltpu.VMEM((CH, H), jnp.float32),
        pltpu.SemaphoreType.DMA,
        pltpu.VMEM_SHARED((NROW, H), jnp.float32),
    ],
)
def _edge_kernel(g_hbm, src_hbm, dst_hbm, zeros_hbm, out_hbm,
                 src_v, dst_v, buf_a, buf_b, gsem, acc):
    c = lax.axis_index("c")
    s = lax.axis_index("s")
    r0 = s * RPT
    _zero_acc(zeros_hbm, buf_a, acc, r0)
    plsc.subcore_barrier()

    for q in range(NGRP):
        pred = (c == CFAST) if q < NGFAST else (c != CFAST)

        @pl.when(pred)
        def _():
            pltpu.sync_copy(src_hbm.at[s, pl.ds(q * GC, GC)], src_v)
            pltpu.sync_copy(dst_hbm.at[s, pl.ds(q * GC, GC)], dst_v)
            pltpu.async_copy(g_hbm.at[src_v.at[0]], buf_a, gsem)

            def body(j2, carry):
                ja = 2 * j2
                jb = ja + 1
                pltpu.make_async_copy(g_hbm.at[src_v.at[ja]], buf_a, gsem).wait()
                pltpu.async_copy(g_hbm.at[src_v.at[jb]], buf_b, gsem)
                pltpu.sync_copy(buf_a, acc.at[dst_v.at[ja]], add=True)
                pltpu.make_async_copy(g_hbm.at[src_v.at[jb]], buf_b, gsem).wait()

                @pl.when(j2 < GC // 2 - 1)
                def _():
                    pltpu.async_copy(g_hbm.at[src_v.at[ja + 2]], buf_a, gsem)

                pltpu.sync_copy(buf_b, acc.at[dst_v.at[jb]], add=True)
                return carry

            lax.fori_loop(0, GC // 2, body, 0)

    plsc.subcore_barrier()
    pltpu.sync_copy(acc.at[pl.ds(r0, RPT)], out_hbm.at[c, pl.ds(r0, RPT)])


@functools.partial(
    pl.kernel,
    out_type=jax.ShapeDtypeStruct((NCORES, NROW, H), jnp.float32),
    mesh=_mesh,
    scratch_types=[
        pltpu.VMEM((NCHUNK, CH), jnp.int32),
        pltpu.VMEM((CH, H), jnp.float32),
        pltpu.VMEM_SHARED((NROW, H), jnp.float32),
    ],
)
def _deg_kernel(dst_hbm, ones_hbm, zeros_hbm, out_hbm, dst_v, ones_v, acc):
    c = lax.axis_index("c")
    s = lax.axis_index("s")
    tg = c * NSUB + s
    r0 = s * RPT
    pltpu.sync_copy(dst_hbm.at[tg], dst_v)
    _zero_acc(zeros_hbm, ones_v, acc, r0)
    pltpu.sync_copy(ones_hbm, ones_v)
    plsc.subcore_barrier()

    def body(j, carry):
        pltpu.sync_copy(ones_v, acc.at[dst_v.at[j]], add=True)
        return carry

    lax.fori_loop(0, NCHUNK, body, 0)
    plsc.subcore_barrier()
    pltpu.sync_copy(acc.at[pl.ds(r0, RPT)], out_hbm.at[c, pl.ds(r0, RPT)])


def _k1_body(x_ref, w_ref, degpart_ref, dinv_ref, g_ref):
    deg = 1.0 + degpart_ref[0, :N, 0] + degpart_ref[1, :N, 0]
    dinv = lax.rsqrt(deg)
    dinv_ref[...] = dinv
    hw = jnp.dot(x_ref[...], w_ref[...], preferred_element_type=jnp.float32)
    g_ref[...] = dinv[:, None] * hw


def _upd_body(spart_ref, g_ref, dinv_ref, b_ref, w_ref, gnext_ref):
    sacc = spart_ref[0, :N, :] + spart_ref[1, :N, :]
    dinv = dinv_ref[...]
    h = jnp.maximum(dinv[:, None] * (sacc + g_ref[...]) + b_ref[...][None, :], 0.0)
    gnext_ref[...] = dinv[:, None] * jnp.dot(
        h, w_ref[...], preferred_element_type=jnp.float32)


def _fin_body(spart_ref, g_ref, dinv_ref, b_ref, batch_ref, wh_ref, bh_ref,
              out_ref):
    sacc = spart_ref[0, :N, :] + spart_ref[1, :N, :]
    dinv = dinv_ref[...]
    ct = dinv[:, None] * (sacc + g_ref[...]) + b_ref[...][None, :]
    gid = lax.broadcasted_iota(jnp.int32, (N, 128), 1)
    oh = (batch_ref[...][:, None] == gid).astype(jnp.float32)
    pooled = lax.dot_general(oh, ct, (((0,), (0,)), ((), ())),
                             preferred_element_type=jnp.float32)
    logits = jnp.dot(pooled, wh_ref[...], preferred_element_type=jnp.float32)
    out_ref[...] = logits[:G, :] + bh_ref[...][None, :]


def kernel(x, edge_index, batch, W1, b1, W2, b2, W3, b3, W4, b4, Wh, bh):
    E = edge_index.shape[1]
    e_pad = NSUB * NCPT * CH
    src = edge_index[0]
    dst = edge_index[1]
    pad = e_pad - E
    src16 = jnp.concatenate([src, jnp.zeros((pad,), jnp.int32)]
                            ).reshape(NSUB, NCPT, CH)
    dst_flat = jnp.concatenate([dst, jnp.full((pad,), N, jnp.int32)])
    dst16 = dst_flat.reshape(NSUB, NCPT, CH)
    dst32 = dst_flat.reshape(NTILES, NCHUNK, CH)
    zerosCH = jnp.zeros((CH, H), jnp.float32)
    onesCH = jnp.ones((CH, H), jnp.float32)

    degpart = _deg_kernel(dst32, onesCH, zerosCH)

    dinv, g1 = pl.pallas_call(
        _k1_body,
        out_shape=(jax.ShapeDtypeStruct((N,), jnp.float32),
                   jax.ShapeDtypeStruct((N, H), jnp.float32)),
    )(x, W1, degpart)

    upd = pl.pallas_call(
        _upd_body,
        out_shape=jax.ShapeDtypeStruct((N, H), jnp.float32),
    )

    s1 = _edge_kernel(g1, src16, dst16, zerosCH)
    g2 = upd(s1, g1, dinv, b1, W2)
    s2 = _edge_kernel(g2, src16, dst16, zerosCH)
    g3 = upd(s2, g2, dinv, b2, W3)
    s3 = _edge_kernel(g3, src16, dst16, zerosCH)
    g4 = upd(s3, g3, dinv, b3, W4)
    s4 = _edge_kernel(g4, src16, dst16, zerosCH)

    logits = pl.pallas_call(
        _fin_body,
        out_shape=jax.ShapeDtypeStruct((G, C), jnp.float32),
    )(s4, g4, dinv, b4, batch, Wh, bh)
    return logits


# speedup vs baseline: 1.0833x; 1.0833x over previous
import functools

import jax
import jax.numpy as jnp
from jax import lax
from jax.experimental import pallas as pl
from jax.experimental.pallas import tpu as pltpu
from jax.experimental.pallas import tpu_sc as plsc

N = 10000
D = 128
H = 128
C = 20
G = 100

NCORES = 2
NSUB = 16
NTILES = NCORES * NSUB
CH = 128
NCPT = 160
NGRP = 10
GC = NCPT // NGRP
NGFAST = 8
CFAST = 1
NCHUNK = 80
NROW = 10112
RPT = NROW // NSUB

_mesh = plsc.VectorSubcoreMesh(core_axis_name="c", subcore_axis_name="s")


def _zero_acc(zeros_hbm, buf, acc, r0):
    pltpu.sync_copy(zeros_hbm, buf)
    for k in range(4):
        pltpu.sync_copy(buf, acc.at[pl.ds(r0 + k * CH, CH)])
    pltpu.sync_copy(buf.at[pl.ds(0, RPT - 4 * CH)],
                    acc.at[pl.ds(r0 + 4 * CH, RPT - 4 * CH)])


@functools.partial(
    pl.kernel,
    out_type=jax.ShapeDtypeStruct((NCORES, NROW, H), jnp.float32),
    mesh=_mesh,
    scratch_types=[
        pltpu.VMEM((GC, CH), jnp.int32),
        pltpu.VMEM((GC, CH), jnp.int32),
        pltpu.VMEM((CH, H), jnp.float32),
        pltpu.VMEM((CH, H), jnp.float32),
        pltpu.SemaphoreType.DMA,
        pltpu.SemaphoreType.DMA,
        pltpu.VMEM_SHARED((NROW, H), jnp.float32),
    ],
)
def _edge_kernel(g_hbm, src_hbm, dst_hbm, zeros_hbm, out_hbm,
                 src_v, dst_v, buf_a, buf_b, gsem, bsem, acc):
    c = lax.axis_index("c")
    s = lax.axis_index("s")
    r0 = s * RPT
    _zero_acc(zeros_hbm, buf_a, acc, r0)
    plsc.subcore_barrier()

    for q in range(NGRP):
        pred = (c == CFAST) if q < NGFAST else (c != CFAST)

        @pl.when(pred)
        def _():
            pltpu.sync_copy(src_hbm.at[s, pl.ds(q * GC, GC)], src_v)
            pltpu.sync_copy(dst_hbm.at[s, pl.ds(q * GC, GC)], dst_v)
            pltpu.async_copy(g_hbm.at[src_v.at[0]], buf_a, gsem)
            pltpu.async_copy(g_hbm.at[src_v.at[1]], buf_b, bsem)

            def body(j2, carry):
                ja = 2 * j2
                jb = ja + 1
                pltpu.make_async_copy(g_hbm.at[src_v.at[ja]], buf_a, gsem).wait()
                pltpu.sync_copy(buf_a, acc.at[dst_v.at[ja]], add=True)

                @pl.when(j2 < GC // 2 - 1)
                def _():
                    pltpu.async_copy(g_hbm.at[src_v.at[ja + 2]], buf_a, gsem)

                pltpu.make_async_copy(g_hbm.at[src_v.at[jb]], buf_b, bsem).wait()
                pltpu.sync_copy(buf_b, acc.at[dst_v.at[jb]], add=True)

                @pl.when(j2 < GC // 2 - 1)
                def _():
                    pltpu.async_copy(g_hbm.at[src_v.at[jb + 2]], buf_b, bsem)

                return carry

            lax.fori_loop(0, GC // 2, body, 0)

    plsc.subcore_barrier()
    pltpu.sync_copy(acc.at[pl.ds(r0, RPT)], out_hbm.at[c, pl.ds(r0, RPT)])


@functools.partial(
    pl.kernel,
    out_type=jax.ShapeDtypeStruct((NCORES, NROW, H), jnp.float32),
    mesh=_mesh,
    scratch_types=[
        pltpu.VMEM((NCHUNK, CH), jnp.int32),
        pltpu.VMEM((CH, H), jnp.float32),
        pltpu.VMEM_SHARED((NROW, H), jnp.float32),
    ],
)
def _deg_kernel(dst_hbm, ones_hbm, zeros_hbm, out_hbm, dst_v, ones_v, acc):
    c = lax.axis_index("c")
    s = lax.axis_index("s")
    tg = c * NSUB + s
    r0 = s * RPT
    pltpu.sync_copy(dst_hbm.at[tg], dst_v)
    _zero_acc(zeros_hbm, ones_v, acc, r0)
    pltpu.sync_copy(ones_hbm, ones_v)
    plsc.subcore_barrier()

    def body(j, carry):
        pltpu.sync_copy(ones_v, acc.at[dst_v.at[j]], add=True)
        return carry

    lax.fori_loop(0, NCHUNK, body, 0)
    plsc.subcore_barrier()
    pltpu.sync_copy(acc.at[pl.ds(r0, RPT)], out_hbm.at[c, pl.ds(r0, RPT)])


def _k1_body(x_ref, w_ref, degpart_ref, dinv_ref, g_ref):
    deg = 1.0 + degpart_ref[0, :N, 0] + degpart_ref[1, :N, 0]
    dinv = lax.rsqrt(deg)
    dinv_ref[...] = dinv
    hw = jnp.dot(x_ref[...], w_ref[...], preferred_element_type=jnp.float32)
    g_ref[...] = dinv[:, None] * hw


def _upd_body(spart_ref, g_ref, dinv_ref, b_ref, w_ref, gnext_ref):
    sacc = spart_ref[0, :N, :] + spart_ref[1, :N, :]
    dinv = dinv_ref[...]
    h = jnp.maximum(dinv[:, None] * (sacc + g_ref[...]) + b_ref[...][None, :], 0.0)
    gnext_ref[...] = dinv[:, None] * jnp.dot(
        h, w_ref[...], preferred_element_type=jnp.float32)


def _fin_body(spart_ref, g_ref, dinv_ref, b_ref, batch_ref, wh_ref, bh_ref,
              out_ref):
    sacc = spart_ref[0, :N, :] + spart_ref[1, :N, :]
    dinv = dinv_ref[...]
    ct = dinv[:, None] * (sacc + g_ref[...]) + b_ref[...][None, :]
    gid = lax.broadcasted_iota(jnp.int32, (N, 128), 1)
    oh = (batch_ref[...][:, None] == gid).astype(jnp.float32)
    pooled = lax.dot_general(oh, ct, (((0,), (0,)), ((), ())),
                             preferred_element_type=jnp.float32)
    logits = jnp.dot(pooled, wh_ref[...], preferred_element_type=jnp.float32)
    out_ref[...] = logits[:G, :] + bh_ref[...][None, :]


def kernel(x, edge_index, batch, W1, b1, W2, b2, W3, b3, W4, b4, Wh, bh):
    E = edge_index.shape[1]
    e_pad = NSUB * NCPT * CH
    src = edge_index[0]
    dst = edge_index[1]
    pad = e_pad - E
    src16 = jnp.concatenate([src, jnp.zeros((pad,), jnp.int32)]
                            ).reshape(NSUB, NCPT, CH)
    dst_flat = jnp.concatenate([dst, jnp.full((pad,), N, jnp.int32)])
    dst16 = dst_flat.reshape(NSUB, NCPT, CH)
    dst32 = dst_flat.reshape(NTILES, NCHUNK, CH)
    zerosCH = jnp.zeros((CH, H), jnp.float32)
    onesCH = jnp.ones((CH, H), jnp.float32)

    degpart = _deg_kernel(dst32, onesCH, zerosCH)

    dinv, g1 = pl.pallas_call(
        _k1_body,
        out_shape=(jax.ShapeDtypeStruct((N,), jnp.float32),
                   jax.ShapeDtypeStruct((N, H), jnp.float32)),
    )(x, W1, degpart)

    upd = pl.pallas_call(
        _upd_body,
        out_shape=jax.ShapeDtypeStruct((N, H), jnp.float32),
    )

    s1 = _edge_kernel(g1, src16, dst16, zerosCH)
    g2 = upd(s1, g1, dinv, b1, W2)
    s2 = _edge_kernel(g2, src16, dst16, zerosCH)
    g3 = upd(s2, g2, dinv, b2, W3)
    s3 = _edge_kernel(g3, src16, dst16, zerosCH)
    g4 = upd(s3, g3, dinv, b3, W4)
    s4 = _edge_kernel(g4, src16, dst16, zerosCH)

    logits = pl.pallas_call(
        _fin_body,
        out_shape=jax.ShapeDtypeStruct((G, C), jnp.float32),
    )(s4, g4, dinv, b4, batch, Wh, bh)
    return logits


# speedup vs baseline: 1.0847x; 1.0013x over previous
import functools

import jax
import jax.numpy as jnp
from jax import lax
from jax.experimental import pallas as pl
from jax.experimental.pallas import tpu as pltpu
from jax.experimental.pallas import tpu_sc as plsc

N = 10000
D = 128
H = 128
C = 20
G = 100

NCORES = 2
NSUB = 16
NTILES = NCORES * NSUB
CH = 128
NCPT = 160
NGRP = 10
GC = NCPT // NGRP
NGFAST = 8
CFAST = 1
NCHUNK = 80
NROW = 10112
RPT = NROW // NSUB

_mesh = plsc.VectorSubcoreMesh(core_axis_name="c", subcore_axis_name="s")


def _zero_acc(zeros_hbm, buf, acc, r0):
    pltpu.sync_copy(zeros_hbm, buf)
    for k in range(4):
        pltpu.sync_copy(buf, acc.at[pl.ds(r0 + k * CH, CH)])
    pltpu.sync_copy(buf.at[pl.ds(0, RPT - 4 * CH)],
                    acc.at[pl.ds(r0 + 4 * CH, RPT - 4 * CH)])


@functools.partial(
    pl.kernel,
    out_type=jax.ShapeDtypeStruct((NCORES, NROW, H), jnp.float32),
    mesh=_mesh,
    scratch_types=[
        pltpu.VMEM((GC, CH), jnp.int32),
        pltpu.VMEM((GC, CH), jnp.int32),
        pltpu.VMEM((CH, H), jnp.float32),
        pltpu.VMEM((CH, H), jnp.float32),
        pltpu.SemaphoreType.DMA,
        pltpu.SemaphoreType.DMA,
        pltpu.VMEM_SHARED((NROW, H), jnp.float32),
    ],
)
def _edge_kernel(g_hbm, src_hbm, dst_hbm, zeros_hbm, out_hbm,
                 src_v, dst_v, buf_a, buf_b, gsem, bsem, acc):
    c = lax.axis_index("c")
    s = lax.axis_index("s")
    r0 = s * RPT
    _zero_acc(zeros_hbm, buf_a, acc, r0)
    plsc.subcore_barrier()

    for q in range(NGRP):
        pred = (c == CFAST) if q < NGFAST else (c != CFAST)

        @pl.when(pred)
        def _():
            pltpu.sync_copy(src_hbm.at[s, pl.ds(q * GC, GC)], src_v)
            pltpu.sync_copy(dst_hbm.at[s, pl.ds(q * GC, GC)], dst_v)
            pltpu.async_copy(g_hbm.at[src_v.at[0]], buf_a, gsem)
            pltpu.async_copy(g_hbm.at[src_v.at[1]], buf_b, bsem)

            def body(j2, carry):
                ja = 2 * j2
                jb = ja + 1
                pltpu.make_async_copy(g_hbm.at[src_v.at[ja]], buf_a, gsem).wait()
                pltpu.sync_copy(buf_a, acc.at[dst_v.at[ja]], add=True)

                @pl.when(j2 < GC // 2 - 1)
                def _():
                    pltpu.async_copy(g_hbm.at[src_v.at[ja + 2]], buf_a, gsem)

                pltpu.make_async_copy(g_hbm.at[src_v.at[jb]], buf_b, bsem).wait()
                pltpu.sync_copy(buf_b, acc.at[dst_v.at[jb]], add=True)

                @pl.when(j2 < GC // 2 - 1)
                def _():
                    pltpu.async_copy(g_hbm.at[src_v.at[jb + 2]], buf_b, bsem)

                return carry

            lax.fori_loop(0, GC // 2, body, 0)

    plsc.subcore_barrier()
    pltpu.sync_copy(acc.at[pl.ds(r0, RPT)], out_hbm.at[c, pl.ds(r0, RPT)])


@functools.partial(
    pl.kernel,
    out_type=jax.ShapeDtypeStruct((NCORES, NROW, H), jnp.float32),
    mesh=_mesh,
    scratch_types=[
        pltpu.VMEM((NCHUNK, CH), jnp.int32),
        pltpu.VMEM((CH, H), jnp.float32),
        pltpu.VMEM_SHARED((NROW, H), jnp.float32),
    ],
)
def _deg_kernel(dst_hbm, ones_hbm, zeros_hbm, out_hbm, dst_v, ones_v, acc):
    c = lax.axis_index("c")
    s = lax.axis_index("s")
    tg = c * NSUB + s
    r0 = s * RPT
    pltpu.sync_copy(dst_hbm.at[tg], dst_v)
    _zero_acc(zeros_hbm, ones_v, acc, r0)
    pltpu.sync_copy(ones_hbm, ones_v)
    plsc.subcore_barrier()

    def body(j, carry):
        pltpu.sync_copy(ones_v, acc.at[dst_v.at[j]], add=True)
        return carry

    lax.fori_loop(0, NCHUNK, body, 0)
    plsc.subcore_barrier()
    pltpu.sync_copy(acc.at[pl.ds(r0, RPT)], out_hbm.at[c, pl.ds(r0, RPT)])


def _k0_body(x_ref, w_ref, hw_ref):
    hw_ref[...] = jnp.dot(x_ref[...], w_ref[...],
                          preferred_element_type=jnp.float32)


def _k1_body(hw_ref, degpart_ref, dinv_ref, g_ref):
    deg = 1.0 + degpart_ref[0, :N, 0] + degpart_ref[1, :N, 0]
    dinv = lax.rsqrt(deg)
    dinv_ref[...] = dinv
    g_ref[...] = dinv[:, None] * hw_ref[...]


def _upd_body(spart_ref, g_ref, dinv_ref, b_ref, w_ref, gnext_ref):
    sacc = spart_ref[0, :N, :] + spart_ref[1, :N, :]
    dinv = dinv_ref[...]
    h = jnp.maximum(dinv[:, None] * (sacc + g_ref[...]) + b_ref[...][None, :], 0.0)
    gnext_ref[...] = dinv[:, None] * jnp.dot(
        h, w_ref[...], preferred_element_type=jnp.float32)


def _fin_body(spart_ref, g_ref, dinv_ref, b_ref, batch_ref, wh_ref, bh_ref,
              out_ref):
    sacc = spart_ref[0, :N, :] + spart_ref[1, :N, :]
    dinv = dinv_ref[...]
    ct = dinv[:, None] * (sacc + g_ref[...]) + b_ref[...][None, :]
    gid = lax.broadcasted_iota(jnp.int32, (N, 128), 1)
    oh = (batch_ref[...][:, None] == gid).astype(jnp.float32)
    pooled = lax.dot_general(oh, ct, (((0,), (0,)), ((), ())),
                             preferred_element_type=jnp.float32)
    logits = jnp.dot(pooled, wh_ref[...], preferred_element_type=jnp.float32)
    out_ref[...] = logits[:G, :] + bh_ref[...][None, :]


def kernel(x, edge_index, batch, W1, b1, W2, b2, W3, b3, W4, b4, Wh, bh):
    E = edge_index.shape[1]
    e_pad = NSUB * NCPT * CH
    src = edge_index[0]
    dst = edge_index[1]
    pad = e_pad - E
    src16 = jnp.concatenate([src, jnp.zeros((pad,), jnp.int32)]
                            ).reshape(NSUB, NCPT, CH)
    dst_flat = jnp.concatenate([dst, jnp.full((pad,), N, jnp.int32)])
    dst16 = dst_flat.reshape(NSUB, NCPT, CH)
    dst32 = dst_flat.reshape(NTILES, NCHUNK, CH)
    zerosCH = jnp.zeros((CH, H), jnp.float32)
    onesCH = jnp.ones((CH, H), jnp.float32)

    hw1 = pl.pallas_call(
        _k0_body,
        out_shape=jax.ShapeDtypeStruct((N, H), jnp.float32),
    )(x, W1)
    degpart = _deg_kernel(dst32, onesCH, zerosCH)

    dinv, g1 = pl.pallas_call(
        _k1_body,
        out_shape=(jax.ShapeDtypeStruct((N,), jnp.float32),
                   jax.ShapeDtypeStruct((N, H), jnp.float32)),
    )(hw1, degpart)

    upd = pl.pallas_call(
        _upd_body,
        out_shape=jax.ShapeDtypeStruct((N, H), jnp.float32),
    )

    s1 = _edge_kernel(g1, src16, dst16, zerosCH)
    g2 = upd(s1, g1, dinv, b1, W2)
    s2 = _edge_kernel(g2, src16, dst16, zerosCH)
    g3 = upd(s2, g2, dinv, b2, W3)
    s3 = _edge_kernel(g3, src16, dst16, zerosCH)
    g4 = upd(s3, g3, dinv, b3, W4)
    s4 = _edge_kernel(g4, src16, dst16, zerosCH)

    logits = pl.pallas_call(
        _fin_body,
        out_shape=jax.ShapeDtypeStruct((G, C), jnp.float32),
    )(s4, g4, dinv, b4, batch, Wh, bh)
    return logits


# speedup vs baseline: 1.0997x; 1.0139x over previous
import functools

import jax
import jax.numpy as jnp
from jax import lax
from jax.experimental import pallas as pl
from jax.experimental.pallas import tpu as pltpu
from jax.experimental.pallas import tpu_sc as plsc

N = 10000
D = 128
H = 128
C = 20
G = 100

NCORES = 2
NSUB = 16
NTILES = NCORES * NSUB
CH = 128
NCPT = 160
NGRP = 5
GC = NCPT // NGRP
NGFAST = 4
CFAST = 1
NCHUNK = 80
NROW = 10112
RPT = NROW // NSUB

_mesh = plsc.VectorSubcoreMesh(core_axis_name="c", subcore_axis_name="s")


def _zero_acc(zeros_hbm, buf, acc, r0):
    pltpu.sync_copy(zeros_hbm, buf)
    for k in range(4):
        pltpu.sync_copy(buf, acc.at[pl.ds(r0 + k * CH, CH)])
    pltpu.sync_copy(buf.at[pl.ds(0, RPT - 4 * CH)],
                    acc.at[pl.ds(r0 + 4 * CH, RPT - 4 * CH)])


@functools.partial(
    pl.kernel,
    out_type=jax.ShapeDtypeStruct((NCORES, NROW, H), jnp.float32),
    mesh=_mesh,
    scratch_types=[
        pltpu.VMEM((GC, CH), jnp.int32),
        pltpu.VMEM((GC, CH), jnp.int32),
        pltpu.VMEM((CH, H), jnp.float32),
        pltpu.VMEM((CH, H), jnp.float32),
        pltpu.SemaphoreType.DMA,
        pltpu.SemaphoreType.DMA,
        pltpu.VMEM_SHARED((NROW, H), jnp.float32),
    ],
)
def _edge_kernel(g_hbm, src_hbm, dst_hbm, zeros_hbm, out_hbm,
                 src_v, dst_v, buf_a, buf_b, gsem, bsem, acc):
    c = lax.axis_index("c")
    s = lax.axis_index("s")
    r0 = s * RPT
    _zero_acc(zeros_hbm, buf_a, acc, r0)
    plsc.subcore_barrier()

    for q in range(NGRP):
        pred = (c == CFAST) if q < NGFAST else (c != CFAST)

        @pl.when(pred)
        def _():
            pltpu.sync_copy(src_hbm.at[s, pl.ds(q * GC, GC)], src_v)
            pltpu.sync_copy(dst_hbm.at[s, pl.ds(q * GC, GC)], dst_v)
            pltpu.async_copy(g_hbm.at[src_v.at[0]], buf_a, gsem)
            pltpu.async_copy(g_hbm.at[src_v.at[1]], buf_b, bsem)

            def body(j2, carry):
                ja = 2 * j2
                jb = ja + 1
                pltpu.make_async_copy(g_hbm.at[src_v.at[ja]], buf_a, gsem).wait()
                pltpu.sync_copy(buf_a, acc.at[dst_v.at[ja]], add=True)

                @pl.when(j2 < GC // 2 - 1)
                def _():
                    pltpu.async_copy(g_hbm.at[src_v.at[ja + 2]], buf_a, gsem)

                pltpu.make_async_copy(g_hbm.at[src_v.at[jb]], buf_b, bsem).wait()
                pltpu.sync_copy(buf_b, acc.at[dst_v.at[jb]], add=True)

                @pl.when(j2 < GC // 2 - 1)
                def _():
                    pltpu.async_copy(g_hbm.at[src_v.at[jb + 2]], buf_b, bsem)

                return carry

            lax.fori_loop(0, GC // 2, body, 0)

    plsc.subcore_barrier()
    pltpu.sync_copy(acc.at[pl.ds(r0, RPT)], out_hbm.at[c, pl.ds(r0, RPT)])


@functools.partial(
    pl.kernel,
    out_type=jax.ShapeDtypeStruct((NCORES, NROW, H), jnp.float32),
    mesh=_mesh,
    scratch_types=[
        pltpu.VMEM((NCHUNK, CH), jnp.int32),
        pltpu.VMEM((CH, H), jnp.float32),
        pltpu.VMEM_SHARED((NROW, H), jnp.float32),
    ],
)
def _deg_kernel(dst_hbm, ones_hbm, zeros_hbm, out_hbm, dst_v, ones_v, acc):
    c = lax.axis_index("c")
    s = lax.axis_index("s")
    tg = c * NSUB + s
    r0 = s * RPT
    pltpu.sync_copy(dst_hbm.at[tg], dst_v)
    _zero_acc(zeros_hbm, ones_v, acc, r0)
    pltpu.sync_copy(ones_hbm, ones_v)
    plsc.subcore_barrier()

    def body(j, carry):
        pltpu.sync_copy(ones_v, acc.at[dst_v.at[j]], add=True)
        return carry

    lax.fori_loop(0, NCHUNK, body, 0)
    plsc.subcore_barrier()
    pltpu.sync_copy(acc.at[pl.ds(r0, RPT)], out_hbm.at[c, pl.ds(r0, RPT)])


def _k0_body(x_ref, w_ref, hw_ref):
    hw_ref[...] = jnp.dot(x_ref[...], w_ref[...],
                          preferred_element_type=jnp.float32)


def _k1_body(hw_ref, degpart_ref, dinv_ref, g_ref):
    deg = 1.0 + degpart_ref[0, :N, 0] + degpart_ref[1, :N, 0]
    dinv = lax.rsqrt(deg)
    dinv_ref[...] = dinv
    g_ref[...] = dinv[:, None] * hw_ref[...]


def _upd_body(spart_ref, g_ref, dinv_ref, b_ref, w_ref, gnext_ref):
    sacc = spart_ref[0, :N, :] + spart_ref[1, :N, :]
    dinv = dinv_ref[...]
    h = jnp.maximum(dinv[:, None] * (sacc + g_ref[...]) + b_ref[...][None, :], 0.0)
    gnext_ref[...] = dinv[:, None] * jnp.dot(
        h, w_ref[...], preferred_element_type=jnp.float32)


def _fin_body(spart_ref, g_ref, dinv_ref, b_ref, batch_ref, wh_ref, bh_ref,
              out_ref):
    sacc = spart_ref[0, :N, :] + spart_ref[1, :N, :]
    dinv = dinv_ref[...]
    ct = dinv[:, None] * (sacc + g_ref[...]) + b_ref[...][None, :]
    gid = lax.broadcasted_iota(jnp.int32, (N, 128), 1)
    oh = (batch_ref[...][:, None] == gid).astype(jnp.float32)
    pooled = lax.dot_general(oh, ct, (((0,), (0,)), ((), ())),
                             preferred_element_type=jnp.float32)
    logits = jnp.dot(pooled, wh_ref[...], preferred_element_type=jnp.float32)
    out_ref[...] = logits[:G, :] + bh_ref[...][None, :]


def kernel(x, edge_index, batch, W1, b1, W2, b2, W3, b3, W4, b4, Wh, bh):
    E = edge_index.shape[1]
    e_pad = NSUB * NCPT * CH
    src = edge_index[0]
    dst = edge_index[1]
    pad = e_pad - E
    src16 = jnp.concatenate([src, jnp.zeros((pad,), jnp.int32)]
                            ).reshape(NSUB, NCPT, CH)
    dst_flat = jnp.concatenate([dst, jnp.full((pad,), N, jnp.int32)])
    dst16 = dst_flat.reshape(NSUB, NCPT, CH)
    dst32 = dst_flat.reshape(NTILES, NCHUNK, CH)
    zerosCH = jnp.zeros((CH, H), jnp.float32)
    onesCH = jnp.ones((CH, H), jnp.float32)

    hw1 = pl.pallas_call(
        _k0_body,
        out_shape=jax.ShapeDtypeStruct((N, H), jnp.float32),
    )(x, W1)
    degpart = _deg_kernel(dst32, onesCH, zerosCH)

    dinv, g1 = pl.pallas_call(
        _k1_body,
        out_shape=(jax.ShapeDtypeStruct((N,), jnp.float32),
                   jax.ShapeDtypeStruct((N, H), jnp.float32)),
    )(hw1, degpart)

    upd = pl.pallas_call(
        _upd_body,
        out_shape=jax.ShapeDtypeStruct((N, H), jnp.float32),
    )

    s1 = _edge_kernel(g1, src16, dst16, zerosCH)
    g2 = upd(s1, g1, dinv, b1, W2)
    s2 = _edge_kernel(g2, src16, dst16, zerosCH)
    g3 = upd(s2, g2, dinv, b2, W3)
    s3 = _edge_kernel(g3, src16, dst16, zerosCH)
    g4 = upd(s3, g3, dinv, b3, W4)
    s4 = _edge_kernel(g4, src16, dst16, zerosCH)

    logits = pl.pallas_call(
        _fin_body,
        out_shape=jax.ShapeDtypeStruct((G, C), jnp.float32),
    )(s4, g4, dinv, b4, batch, Wh, bh)
    return logits
